# baseline (speedup 1.0000x reference)
import functools

import jax
import jax.numpy as jnp
from jax import lax
from jax.experimental import pallas as pl
from jax.experimental.pallas import tpu as pltpu
from jax.experimental.pallas import tpu_sc as plsc

TIMESTEPS = 1000
BATCH = 16384
LANES = 16


def _lookup_body(chunk, t_hbm, gamma_hbm, out_hbm, table_v, t_v, out_v,
                 sem_tbl, sem_t):
    nc = lax.axis_size("c")
    wid = lax.axis_index("s") * nc + lax.axis_index("c")
    base = wid * chunk
    tbl_copy = pltpu.async_copy(gamma_hbm, table_v, sem_tbl)
    t_copy = pltpu.async_copy(t_hbm.at[pl.ds(base, chunk)], t_v, sem_t)
    tbl_copy.wait()
    t_copy.wait()
    @plsc.parallel_loop(0, chunk, step=LANES, unroll=8)
    def _body(off):
        tv = t_v[pl.ds(off, LANES)]
        x = tv * jnp.float32(TIMESTEPS)
        xi = x.astype(jnp.int32)
        frac = x - xi.astype(jnp.float32)
        up = (frac > 0.5) | ((frac == 0.5) & ((xi & 1) == 1))
        idx = jnp.where(up, xi + 1, xi)
        out_v[pl.ds(off, LANES)] = plsc.load_gather(table_v, [idx])
    pltpu.sync_copy(out_v, out_hbm.at[pl.ds(base, chunk)])


def kernel(t, gamma):
    info = plsc.get_sparse_core_info()
    nw = info.num_cores * info.num_subcores
    chunk = BATCH // nw
    table = gamma.shape[0]

    mesh = plsc.VectorSubcoreMesh(core_axis_name="c", subcore_axis_name="s")
    k = functools.partial(
        pl.kernel,
        mesh=mesh,
        out_type=jax.ShapeDtypeStruct((BATCH,), jnp.float32),
        scratch_types=[
            pltpu.VMEM((table,), jnp.float32),
            pltpu.VMEM((chunk,), jnp.float32),
            pltpu.VMEM((chunk,), jnp.float32),
            pltpu.SemaphoreType.DMA,
            pltpu.SemaphoreType.DMA,
        ],
        compiler_params=pltpu.CompilerParams(needs_layout_passes=False),
    )(functools.partial(_lookup_body, chunk))
    return k(t, gamma)


# speedup vs baseline: 1.0802x; 1.0802x over previous
import functools

import jax
import jax.numpy as jnp
from jax import lax
from jax.experimental import pallas as pl
from jax.experimental.pallas import tpu as pltpu
from jax.experimental.pallas import tpu_sc as plsc

TIMESTEPS = 1000
BATCH = 16384
LANES = 16


def _lookup_body(chunk, t_hbm, gamma_hbm, out_hbm, table_v, t_v, out_v,
                 sem_tbl, sem_t):
    nc = lax.axis_size("c")
    wid = lax.axis_index("s") * nc + lax.axis_index("c")
    base = wid * chunk
    tbl_copy = pltpu.async_copy(gamma_hbm, table_v, sem_tbl)
    t_copy = pltpu.async_copy(t_hbm.at[pl.ds(base, chunk)], t_v, sem_t)
    tbl_copy.wait()
    t_copy.wait()
    @plsc.parallel_loop(0, chunk, step=LANES, unroll=8)
    def _body(off):
        tv = t_v[pl.ds(off, LANES)]
        x = tv * jnp.float32(TIMESTEPS)
        xi = x.astype(jnp.int32)
        frac = x - xi.astype(jnp.float32)
        up = (frac > 0.5) | ((frac == 0.5) & ((xi & 1) == 1))
        idx = jnp.where(up, xi + 1, xi)
        out_v[pl.ds(off, LANES)] = plsc.load_gather(table_v, [idx])
    pltpu.sync_copy(out_v, out_hbm.at[pl.ds(base, chunk)])


def kernel(t, gamma):
    info = plsc.get_sparse_core_info()
    nw = 1 * info.num_subcores
    chunk = BATCH // nw
    table = gamma.shape[0]

    mesh = plsc.VectorSubcoreMesh(core_axis_name="c", subcore_axis_name="s",
                                  num_cores=1)
    k = functools.partial(
        pl.kernel,
        mesh=mesh,
        out_type=jax.ShapeDtypeStruct((BATCH,), jnp.float32),
        scratch_types=[
            pltpu.VMEM((table,), jnp.float32),
            pltpu.VMEM((chunk,), jnp.float32),
            pltpu.VMEM((chunk,), jnp.float32),
            pltpu.SemaphoreType.DMA,
            pltpu.SemaphoreType.DMA,
        ],
        compiler_params=pltpu.CompilerParams(needs_layout_passes=False),
    )(functools.partial(_lookup_body, chunk))
    return k(t, gamma)


# speedup vs baseline: 1.0806x; 1.0004x over previous
import functools

import jax
import jax.numpy as jnp
from jax import lax
from jax.experimental import pallas as pl
from jax.experimental.pallas import tpu as pltpu
from jax.experimental.pallas import tpu_sc as plsc

TIMESTEPS = 1000
BATCH = 16384
LANES = 16


def _lookup_body(chunk, t_hbm, gamma_hbm, out_hbm, table_v, t_v, out_v,
                 sem_tbl, sem_t):
    nc = lax.axis_size("c")
    wid = lax.axis_index("s") * nc + lax.axis_index("c")
    base = wid * chunk
    tbl_copy = pltpu.async_copy(gamma_hbm, table_v, sem_tbl)
    t_copy = pltpu.async_copy(t_hbm.at[pl.ds(base, chunk)], t_v, sem_t)
    tbl_copy.wait()
    t_copy.wait()
    @plsc.parallel_loop(0, chunk, step=LANES, unroll=8)
    def _body(off):
        tv = t_v[pl.ds(off, LANES)]
        x = tv * jnp.float32(TIMESTEPS)
        xi = x.astype(jnp.int32)
        frac = x - xi.astype(jnp.float32)
        up = (frac > 0.5) | ((frac == 0.5) & ((xi & 1) == 1))
        idx = jnp.where(up, xi + 1, xi)
        out_v[pl.ds(off, LANES)] = plsc.load_gather(table_v, [idx])
    pltpu.sync_copy(out_v, out_hbm.at[pl.ds(base, chunk)])


def kernel(t, gamma):
    info = plsc.get_sparse_core_info()
    nw = 8
    chunk = BATCH // nw
    table = gamma.shape[0]

    mesh = plsc.VectorSubcoreMesh(core_axis_name="c", subcore_axis_name="s",
                                  num_cores=1, num_subcores=8)
    k = functools.partial(
        pl.kernel,
        mesh=mesh,
        out_type=jax.ShapeDtypeStruct((BATCH,), jnp.float32),
        scratch_types=[
            pltpu.VMEM((table,), jnp.float32),
            pltpu.VMEM((chunk,), jnp.float32),
            pltpu.VMEM((chunk,), jnp.float32),
            pltpu.SemaphoreType.DMA,
            pltpu.SemaphoreType.DMA,
        ],
        compiler_params=pltpu.CompilerParams(needs_layout_passes=False),
    )(functools.partial(_lookup_body, chunk))
    return k(t, gamma)
